# Initial kernel scaffold; baseline (speedup 1.0000x reference)
import jax
import jax.numpy as jnp
from jax.experimental import pallas as pl


def kernel(inputs, params):
    raise NotImplementedError("write your pallas kernel here")


# speedup vs baseline: 4.9681x; 4.9681x over previous
import functools

import jax
import jax.numpy as jnp
from jax.experimental import pallas as pl

_N = 10000
_NP = 10240
_K = 3
_Q = 256
_PREC = jax.lax.Precision.HIGHEST
_INTERPRET = False


def _dense3_body(x_ref, w1, b1, w2, b2, w3, b3, o_ref, *, relus):
    x = x_ref[...]
    for w, b, r in ((w1, b1, relus[0]), (w2, b2, relus[1]), (w3, b3, relus[2])):
        x = jnp.dot(x, w[...], precision=_PREC) + b[...]
        if r:
            x = jnp.maximum(x, 0.0)
    o_ref[...] = x


def _dense3(x, p1, p2, p3, relus):
    co = p3["W"].shape[1]
    return pl.pallas_call(
        functools.partial(_dense3_body, relus=relus),
        out_shape=jax.ShapeDtypeStruct((_NP, co), jnp.float32),
        interpret=_INTERPRET,
    )(x, p1["W"], p1["b"].reshape(1, -1), p2["W"], p2["b"].reshape(1, -1),
      p3["W"], p3["b"].reshape(1, -1))


def _sh_body(x_ref, w_ref, b_ref, o_ref):
    o_ref[...] = jnp.dot(x_ref[...], w_ref[...], precision=_PREC) + b_ref[...]


def _conv_body(xq_ref, shq_ref, st_ref, w1_ref, w2_ref, b2_ref, o_ref):
    st = st_ref[...]
    s0 = st[0:1, :]
    s1 = st[1:2, :]
    s2 = st[2:3, :]
    h_row = st[3:4, :]
    sq_row = s0 * s0 + s1 * s1 + s2 * s2
    shq = shq_ref[...]
    q0 = shq[:, 0:1]
    q1 = shq[:, 1:2]
    q2 = shq[:, 2:3]
    qq = q0 * q0 + q1 * q1 + q2 * q2
    cmask = (jax.lax.broadcasted_iota(jnp.int32, (1, 8), 1) < 3).astype(jnp.float32)
    cross = jnp.dot(shq * cmask, st, precision=_PREC)
    d2 = (qq + sq_row) - 2.0 * cross
    jidx = jax.lax.broadcasted_iota(jnp.int32, (1, _NP), 1).astype(jnp.float32)
    d2 = jnp.where(jidx < float(_N), d2, jnp.inf)
    msgs = []
    for _ in range(_K):
        m = jnp.min(d2, axis=1, keepdims=True)
        jsel = jnp.min(jnp.where(d2 == m, jidx, float(_NP)),
                       axis=1, keepdims=True)
        selm = jidx == jsel
        hsel = jnp.sum(jnp.where(selm, h_row, 0.0),
                       axis=1, keepdims=True)
        d2 = jnp.where(selm, jnp.inf, d2)
        msgs.append(hsel * jnp.exp(-10.0 * jnp.maximum(m, 0.0)))
    mean = (msgs[0] + msgs[1] + msgs[2]) / 3.0
    mx = jnp.maximum(jnp.maximum(msgs[0], msgs[1]), msgs[2])
    o_ref[...] = (jnp.dot(xq_ref[...], w1_ref[...], precision=_PREC)
                  + (mean * w2_ref[0:1, :] + mx * w2_ref[1:2, :])) + b2_ref[...]


def _conv(x, p):
    c = x.shape[1]
    co = p["lin_out1"]["W"].shape[1]
    wsh = jnp.zeros((c, 8), jnp.float32)
    wsh = wsh.at[:, 0:3].set(p["lin_s"]["W"]).at[:, 3:4].set(p["lin_h"]["W"])
    bsh = jnp.zeros((1, 8), jnp.float32)
    bsh = bsh.at[0, 0:3].set(p["lin_s"]["b"]).at[0, 3].set(p["lin_h"]["b"][0])
    sh = pl.pallas_call(
        _sh_body,
        out_shape=jax.ShapeDtypeStruct((_NP, 8), jnp.float32),
        interpret=_INTERPRET,
    )(x, wsh, bsh)
    st = sh.T
    grid = (_NP // _Q,)
    return pl.pallas_call(
        _conv_body,
        grid=grid,
        in_specs=[
            pl.BlockSpec((_Q, c), lambda i: (i, 0)),
            pl.BlockSpec((_Q, 8), lambda i: (i, 0)),
            pl.BlockSpec((8, _NP), lambda i: (0, 0)),
            pl.BlockSpec((c, co), lambda i: (0, 0)),
            pl.BlockSpec((2, co), lambda i: (0, 0)),
            pl.BlockSpec((1, co), lambda i: (0, 0)),
        ],
        out_specs=pl.BlockSpec((_Q, co), lambda i: (i, 0)),
        out_shape=jax.ShapeDtypeStruct((_NP, co), jnp.float32),
        interpret=_INTERPRET,
    )(x, sh, st, p["lin_out1"]["W"], p["lin_out2"]["W"],
      p["lin_out2"]["b"].reshape(1, -1))


def kernel(inputs, params):
    x = inputs.reshape(inputs.shape[0], -1)
    x = jnp.pad(x, ((0, _NP - _N), (0, 0)))
    p = params
    x = _dense3(x, p["dense1"], p["dense2"], p["dense3"], (False, True, True))
    for name in ("mp1", "mp2", "mp3", "mp4", "mp5", "mp6"):
        x = _conv(x, p[name])
    x = _dense3(x, p["dense4"], p["dense5"], p["dense6"], (True, True, True))
    return x[:_N]
